# baseline (speedup 1.0000x reference)
import functools

import jax
import jax.numpy as jnp
from jax import lax
from jax.experimental import pallas as pl
from jax.experimental.pallas import tpu as pltpu
from jax.experimental.pallas import tpu_sc as plsc

NC, NS, L = 2, 16, 16
NW = NC * NS
D = 128
DW = 144
CHUNK = 128
FFN = 512


def _rup(x, m):
    return (x + m - 1) // m * m


def _pad1(x, n, val):
    if x.shape[0] == n:
        return x
    return jnp.concatenate([x, jnp.full((n - x.shape[0],), val, x.dtype)])


@functools.partial(jax.jit, static_argnums=(2,))
def _sc_gather(table, idx, n_chunks):
    B = idx.shape[0]
    bpw = B // NW
    CW = bpw // n_chunks
    Dm = table.shape[1]
    mesh = plsc.VectorSubcoreMesh(core_axis_name="c", subcore_axis_name="s",
                                  num_cores=NC, num_subcores=NS)

    @functools.partial(
        pl.kernel,
        out_type=jax.ShapeDtypeStruct((B, Dm), jnp.float32),
        mesh=mesh,
        compiler_params=pltpu.CompilerParams(use_tc_tiling_on_sc=False, needs_layout_passes=False),
        scratch_types=[
            pltpu.VMEM((CW,), jnp.int32),
            pltpu.VMEM((CW, Dm), jnp.float32),
            pltpu.SemaphoreType.DMA,
        ],
    )
    def gk(tab_hbm, idx_hbm, out_hbm, idx_v, rows_v, sem):
        wid = lax.axis_index("s") * NC + lax.axis_index("c")
        base = wid * bpw
        for i in range(n_chunks):
            off = base + i * CW
            pltpu.sync_copy(idx_hbm.at[pl.ds(off, CW)], idx_v)
            pltpu.async_copy(tab_hbm.at[idx_v], rows_v, sem).wait()
            pltpu.sync_copy(rows_v, out_hbm.at[pl.ds(off, CW)])

    return gk(table, idx)


def _make_edge_kernel(E_pad, Nsrc16, NdP, C, NR):
    E_half = E_pad // NC
    n_t = E_half // NS
    n_chunks = n_t // C
    K = n_chunks // NR
    R = NdP + 16
    rows_per_tile = NdP // NS
    zr = R // NS
    mesh = plsc.VectorSubcoreMesh(core_axis_name="c", subcore_axis_name="s",
                                  num_cores=NC, num_subcores=NS)

    scratch = [
        pltpu.VMEM((Nsrc16,), jnp.float32),
        pltpu.VMEM((NdP + 16,), jnp.float32),
        pltpu.VMEM((16,), jnp.float32),
        pltpu.VMEM((16,), jnp.float32),
        pltpu.VMEM((C,), jnp.float32),
        pltpu.VMEM_SHARED((R, DW), jnp.float32),
    ]
    scratch += [pltpu.VMEM((C,), jnp.int32) for _ in range(3 * NR)]
    scratch += [pltpu.VMEM((C, DW), jnp.float32) for _ in range(NR)]
    scratch += [pltpu.SemaphoreType.DMA for _ in range(2 * NR)]

    @functools.partial(
        pl.kernel,
        out_type=jax.ShapeDtypeStruct((NC, NdP, DW), jnp.float32),
        mesh=mesh,
        compiler_params=pltpu.CompilerParams(use_tc_tiling_on_sc=False,
                                             needs_layout_passes=False),
        scratch_types=scratch,
    )
    def ek(*refs):
        (src_hbm, dst_hbm, tf_hbm, zp_hbm, zs_hbm, zd_hbm, ea_hbm, stab_hbm,
         out_hbm, zs_tab, zd_tab, ea_tab, stab_v, ex_v, u_sh) = refs[:15]
        srcb = refs[15:15 + NR]
        dstb = refs[15 + NR:15 + 2 * NR]
        tfb = refs[15 + 2 * NR:15 + 3 * NR]
        rows = refs[15 + 3 * NR:15 + 4 * NR]
        gsem = refs[15 + 4 * NR:15 + 5 * NR]
        ssem = refs[15 + 5 * NR:15 + 6 * NR]
        cid = lax.axis_index("c")
        sid = lax.axis_index("s")

        pltpu.sync_copy(zs_hbm, zs_tab)
        pltpu.sync_copy(zd_hbm, zd_tab)
        pltpu.sync_copy(ea_hbm, ea_tab)
        pltpu.sync_copy(stab_hbm, stab_v)
        stab = stab_v[...]

        def zrow(i, _):
            for j in range(DW // L):
                rows[0][i, pl.ds(j * L, L)] = jnp.zeros((L,), jnp.float32)
            return 0
        lax.fori_loop(0, C, zrow, 0)
        zbase = sid * zr
        off = 0
        while off < zr:
            n = min(C, zr - off)
            pltpu.sync_copy(rows[0].at[pl.ds(0, n)],
                            u_sh.at[pl.ds(zbase + off, n)])
            off += n
        plsc.subcore_barrier()

        ebase = cid * E_half + sid * n_t

        def arm(i, b):
            eoff = ebase + i * C
            pltpu.sync_copy(src_hbm.at[pl.ds(eoff, C)], srcb[b])
            pltpu.sync_copy(dst_hbm.at[pl.ds(eoff, C)], dstb[b])
            pltpu.sync_copy(tf_hbm.at[pl.ds(eoff, C)], tfb[b])
            pltpu.async_copy(zp_hbm.at[srcb[b]], rows[b], gsem[b])

        def wait_scatter(b):
            del b

        def proc(b):
            pltpu.make_async_copy(zp_hbm.at[srcb[b]], rows[b],
                                  gsem[b]).wait()

            def grp(g, _):
                sl = pl.ds(g * L, L)
                lg = (plsc.load_gather(zs_tab, [srcb[b][sl]])
                      + plsc.load_gather(zd_tab, [dstb[b][sl]])
                      + plsc.load_gather(ea_tab, [tfb[b][sl]]))
                lg = jnp.maximum(lg, 0.2 * lg)
                ex_v[sl] = jnp.exp(lg - stab)
                return 0
            lax.fori_loop(0, C // L, grp, 0)

            def rowmul(g, _):
                exg = ex_v[pl.ds(g * L, L)]
                for lane in range(L):
                    s = exg[lane]
                    e = g * L + lane
                    for j in range(DW // L):
                        sl = pl.ds(j * L, L)
                        rows[b][e, sl] = rows[b][e, sl] * s
                return 0
            lax.fori_loop(0, C // L, rowmul, 0)
            pltpu.sync_copy(rows[b], u_sh.at[dstb[b]], add=True)

        for b in range(NR - 1):
            arm(b, b)

        def body(k, _):
            for j in range(NR):
                i = NR * k + j
                proc(j)
                t = i + NR - 1
                bt = (j - 1) % NR
                if j == 0:
                    @pl.when(k > 0)
                    def _():
                        wait_scatter(bt)
                    arm(t, bt)
                else:
                    @pl.when(k < K - 1)
                    def _():
                        wait_scatter(bt)
                        arm(t, bt)
            return 0
        lax.fori_loop(0, K, body, 0)

        for b in range(NR):
            wait_scatter(b)
        plsc.subcore_barrier()
        ob = sid * rows_per_tile
        pltpu.sync_copy(u_sh.at[pl.ds(ob, rows_per_tile)],
                        out_hbm.at[cid, pl.ds(ob, rows_per_tile)])

    return ek


def _prep(h_src, h_dst, W, Wd, a_s, a_d, etab, a_e, Ns16, Ndt):
    Ns = h_src.shape[0]
    Nd = h_dst.shape[0]
    T = etab.shape[0]

    def body(hs_ref, hd_ref, w_ref, wd_ref, as_ref, ad_ref, te_ref, ae_ref,
             zp_ref, zs_ref, zd_ref, ea_ref, st_ref):
        z = jnp.dot(hs_ref[...], w_ref[...], preferred_element_type=jnp.float32)
        zp_ref[...] = jnp.zeros((Ns16, DW), jnp.float32)
        zp_ref[0:Ns, 0:D] = z
        zp_ref[0:Ns, D:D + 1] = jnp.ones((Ns, 1), jnp.float32)
        zsv = jnp.dot(z, as_ref[...], preferred_element_type=jnp.float32)
        zs_ref[...] = jnp.zeros((Ns16, 1), jnp.float32)
        zs_ref[0:Ns, :] = zsv
        wdv = jnp.dot(wd_ref[...], ad_ref[...], preferred_element_type=jnp.float32)
        zdv = jnp.dot(hd_ref[...], wdv, preferred_element_type=jnp.float32)
        zd_ref[...] = jnp.zeros((Ndt, 1), jnp.float32)
        zd_ref[0:Nd, :] = zdv
        eav = jnp.dot(te_ref[...], ae_ref[...], preferred_element_type=jnp.float32)
        ea_ref[...] = jnp.zeros((1, 16), jnp.float32)
        ea_ref[0:1, 0:T] = jnp.reshape(eav, (1, T))
        m = jnp.max(zsv) + jnp.max(zdv) + jnp.max(eav)
        m = jnp.maximum(m, 0.2 * m)
        st_ref[...] = jnp.full((1, 16), m, jnp.float32)

    zp, zs, zd, ea, st = pl.pallas_call(
        body,
        out_shape=[
            jax.ShapeDtypeStruct((Ns16, DW), jnp.float32),
            jax.ShapeDtypeStruct((Ns16, 1), jnp.float32),
            jax.ShapeDtypeStruct((Ndt, 1), jnp.float32),
            jax.ShapeDtypeStruct((1, 16), jnp.float32),
            jax.ShapeDtypeStruct((1, 16), jnp.float32),
        ],
    )(h_src, h_dst, W, Wd, a_s.reshape(D, 1), a_d.reshape(D, 1), etab,
      a_e.reshape(-1, 1))
    return zp, zs.reshape(-1), zd.reshape(-1), ea.reshape(-1), st.reshape(-1)


def _epilogue(up, h_dst, W1, W2):
    Nd = h_dst.shape[0]
    Nd16 = up.shape[1]
    BR = min(2048, Nd16)
    grid = (pl.cdiv(Nd16, BR),)

    def body(up_ref, hd_ref, w1_ref, w2_ref, out_ref):
        u = up_ref[0] + up_ref[1]
        den = u[:, D:D + 1]
        safe = jnp.where(den > 0, den, 1.0)
        agg = jnp.where(den > 0, u[:, 0:D] / safe, 0.0)
        h = jnp.where(agg > 0, agg, jnp.exp(jnp.minimum(agg, 0.0)) - 1.0)
        hf = jnp.dot(jnp.maximum(jnp.dot(h, w1_ref[...],
                                         preferred_element_type=jnp.float32),
                                 0.0),
                     w2_ref[...], preferred_element_type=jnp.float32)
        out_ref[...] = hd_ref[...] + h + hf

    return pl.pallas_call(
        body,
        grid=grid,
        in_specs=[
            pl.BlockSpec((2, BR, DW), lambda i: (0, i, 0)),
            pl.BlockSpec((BR, D), lambda i: (i, 0)),
            pl.BlockSpec((D, FFN), lambda i: (0, 0)),
            pl.BlockSpec((FFN, D), lambda i: (0, 0)),
        ],
        out_specs=pl.BlockSpec((BR, D), lambda i: (i, 0)),
        out_shape=jax.ShapeDtypeStruct((Nd, D), jnp.float32),
    )(up, h_dst, W1, W2)


def _matmul_tc(x, w):
    def body(x_ref, w_ref, o_ref):
        o_ref[...] = jnp.dot(x_ref[...], w_ref[...],
                             preferred_element_type=jnp.float32)
    return pl.pallas_call(
        body,
        out_shape=jax.ShapeDtypeStruct((x.shape[0], w.shape[1]), jnp.float32),
    )(x, w)


def _head_tc(x, wh, bh):
    def body(x_ref, w_ref, b_ref, o_ref):
        y = jnp.dot(x_ref[...], w_ref[...], preferred_element_type=jnp.float32)
        o_ref[...] = 1.0 / (1.0 + jnp.exp(-(y + b_ref[...])))
    return pl.pallas_call(
        body,
        out_shape=jax.ShapeDtypeStruct((x.shape[0], wh.shape[1]), jnp.float32),
    )(x, wh, bh.reshape(1, -1))


def _gat_layer(h_src, h_dst, srcp, dstp, tfp, etab, p, edge_k, Ns16, Ndt):
    zp, zs, zd, ea, st = _prep(h_src, h_dst, p['W'], p['Wd'], p['a_s'],
                               p['a_d'], etab, p['a_e'], Ns16, Ndt)
    up = edge_k(srcp, dstp, tfp, zp, zs, zd, ea, st)
    return _epilogue(up, h_dst, p['W1'], p['W2'])


def kernel(wid, ws_src, ws_dst, tffrac, ww_src, ww_dst, tffrac_ww,
           ss_src, ss_dst, simfrac, sent_init, embed_table, tf_embed,
           sim_embed, W_proj, p_w2s, p_s2w, p_s2s, p_w2w, Wh, bh):
    N_W = wid.shape[0]
    N_S = sent_init.shape[0]
    E_WS = ws_src.shape[0]
    E_WW = ww_src.shape[0]
    E_SS = ss_src.shape[0]
    NW16 = _rup(N_W, 16)
    NS16 = _rup(N_S, 16)
    NW_P = _rup(N_W, 128)
    NS_P = _rup(N_S, 128)

    EP_WS = _rup(E_WS, NW * CHUNK * 3)
    EP_WW = _rup(E_WW, NW * 64 * 2)
    EP_SS = _rup(E_SS, NW * CHUNK * 3)
    i32 = jnp.int32
    ws_s = _pad1(ws_src.astype(i32), EP_WS, 0)
    ws_d = _pad1(ws_dst.astype(i32), EP_WS, NS_P)
    ws_d_rev = _pad1(ws_dst.astype(i32), EP_WS, 0)
    ws_s_rev = _pad1(ws_src.astype(i32), EP_WS, NW_P)
    tf_p = _pad1(tffrac.astype(i32), EP_WS, 0)
    ww_s = _pad1(ww_src.astype(i32), EP_WW, 0)
    ww_d = _pad1(ww_dst.astype(i32), EP_WW, NW_P)
    tfw_p = _pad1(tffrac_ww.astype(i32), EP_WW, 0)
    ss_s = _pad1(ss_src.astype(i32), EP_SS, 0)
    ss_d = _pad1(ss_dst.astype(i32), EP_SS, NS_P)
    sim_p = _pad1(simfrac.astype(i32), EP_SS, 0)

    B = _rup(N_W, NW * 64)
    widp = _pad1(wid.astype(i32), B, 0)
    word_feature = _sc_gather(embed_table, widp, (B // NW) // 64)[:N_W]

    sent_feature = _matmul_tc(sent_init, W_proj)

    ek_w2s = _make_edge_kernel(EP_WS, NW16, NS_P, CHUNK, 3)
    ek_w2w = _make_edge_kernel(EP_WW, NW16, NW_P, 64, 2)
    ek_s2w = _make_edge_kernel(EP_WS, NS16, NW_P, 64, 3)
    ek_s2s = _make_edge_kernel(EP_SS, NS16, NS_P, CHUNK, 3)

    word_state = word_feature
    sent_state = _gat_layer(word_state, sent_feature, ws_s, ws_d, tf_p,
                            tf_embed, p_w2s, ek_w2s, NW16, NS_P + 16)
    word_state = _gat_layer(word_state, word_state, ww_s, ww_d, tfw_p,
                            tf_embed, p_w2w, ek_w2w, NW16, NW_P + 16)
    word_state = _gat_layer(sent_state, word_state, ws_d_rev, ws_s_rev, tf_p,
                            tf_embed, p_s2w, ek_s2w, NS16, NW_P + 16)
    sent_state = _gat_layer(word_state, sent_state, ws_s, ws_d, tf_p,
                            tf_embed, p_w2s, ek_w2s, NW16, NS_P + 16)
    sent_state = _gat_layer(sent_state, sent_state, ss_s, ss_d, sim_p,
                            sim_embed, p_s2s, ek_s2s, NS16, NS_P + 16)

    return _head_tc(sent_state, Wh, bh)


# speedup vs baseline: 1.5731x; 1.5731x over previous
import functools

import jax
import jax.numpy as jnp
from jax import lax
from jax.experimental import pallas as pl
from jax.experimental.pallas import tpu as pltpu
from jax.experimental.pallas import tpu_sc as plsc

NC, NS, L = 2, 16, 16
NW = NC * NS
D = 128
DW = 144
CHUNK = 128
FFN = 512


def _rup(x, m):
    return (x + m - 1) // m * m


def _pad1(x, n, val):
    if x.shape[0] == n:
        return x
    return jnp.concatenate([x, jnp.full((n - x.shape[0],), val, x.dtype)])


@functools.partial(jax.jit, static_argnums=(2,))
def _sc_gather(table, idx, n_chunks):
    B = idx.shape[0]
    bpw = B // NW
    CW = bpw // n_chunks
    Dm = table.shape[1]
    mesh = plsc.VectorSubcoreMesh(core_axis_name="c", subcore_axis_name="s",
                                  num_cores=NC, num_subcores=NS)

    @functools.partial(
        pl.kernel,
        out_type=jax.ShapeDtypeStruct((B, Dm), jnp.float32),
        mesh=mesh,
        compiler_params=pltpu.CompilerParams(use_tc_tiling_on_sc=False, needs_layout_passes=False),
        scratch_types=[
            pltpu.VMEM((CW,), jnp.int32),
            pltpu.VMEM((CW, Dm), jnp.float32),
            pltpu.SemaphoreType.DMA,
        ],
    )
    def gk(tab_hbm, idx_hbm, out_hbm, idx_v, rows_v, sem):
        wid = lax.axis_index("s") * NC + lax.axis_index("c")
        base = wid * bpw
        for i in range(n_chunks):
            off = base + i * CW
            pltpu.sync_copy(idx_hbm.at[pl.ds(off, CW)], idx_v)
            pltpu.async_copy(tab_hbm.at[idx_v], rows_v, sem).wait()
            pltpu.sync_copy(rows_v, out_hbm.at[pl.ds(off, CW)])

    return gk(table, idx)


def _make_edge_kernel(E_pad, Nsrc16, NdP, C):
    E_half = E_pad // NC
    n_t = E_half // NS
    n_chunks = n_t // C
    n_pairs = n_chunks // 2
    R = NdP + 16
    rows_per_tile = NdP // NS
    zr = R // NS
    mesh = plsc.VectorSubcoreMesh(core_axis_name="c", subcore_axis_name="s",
                                  num_cores=NC, num_subcores=NS)

    @functools.partial(
        pl.kernel,
        out_type=jax.ShapeDtypeStruct((NC, NdP, DW), jnp.float32),
        mesh=mesh,
        compiler_params=pltpu.CompilerParams(use_tc_tiling_on_sc=False,
                                             needs_layout_passes=False),
        scratch_types=[
            pltpu.VMEM((Nsrc16,), jnp.float32),
            pltpu.VMEM((NdP + 16,), jnp.float32),
            pltpu.VMEM((16,), jnp.float32),
            pltpu.VMEM((16,), jnp.float32),
            pltpu.VMEM((3, C), jnp.int32),
            pltpu.VMEM((3, C), jnp.int32),
            pltpu.VMEM((C, DW), jnp.float32),
            pltpu.VMEM((C, DW), jnp.float32),
            pltpu.VMEM((C,), jnp.float32),
            pltpu.VMEM_SHARED((R, DW), jnp.float32),
            pltpu.SemaphoreType.DMA,
            pltpu.SemaphoreType.DMA,
        ],
    )
    def ek(ipk, zp_hbm, zs_hbm, zd_hbm, ea_hbm, stab_hbm, out_hbm,
           zs_tab, zd_tab, ea_tab, stab_v, idxA, idxB, rowsA, rowsB,
           ex_v, u_sh, gsemA, gsemB):
        cid = lax.axis_index("c")
        sid = lax.axis_index("s")

        pltpu.sync_copy(zs_hbm, zs_tab)
        pltpu.sync_copy(zd_hbm, zd_tab)
        pltpu.sync_copy(ea_hbm, ea_tab)
        pltpu.sync_copy(stab_hbm, stab_v)
        stab = stab_v[...]

        def zrow(i, _):
            for j in range(DW // L):
                rowsA[i, pl.ds(j * L, L)] = jnp.zeros((L,), jnp.float32)
            return 0
        lax.fori_loop(0, C, zrow, 0)
        zbase = sid * zr
        off = 0
        while off < zr:
            n = min(C, zr - off)
            pltpu.sync_copy(rowsA.at[pl.ds(0, n)],
                            u_sh.at[pl.ds(zbase + off, n)])
            off += n
        plsc.subcore_barrier()

        cbase = (cid * NS + sid) * n_chunks

        def compute(idx, rows):
            def grp(g, _):
                sl = pl.ds(g * L, L)
                lg = (plsc.load_gather(zs_tab, [idx[0, sl]])
                      + plsc.load_gather(zd_tab, [idx[1, sl]])
                      + plsc.load_gather(ea_tab, [idx[2, sl]]))
                lg = jnp.maximum(lg, 0.2 * lg)
                ex_v[sl] = jnp.exp(lg - stab)
                return 0
            lax.fori_loop(0, C // L, grp, 0)

            def rowmul(g, _):
                exg = ex_v[pl.ds(g * L, L)]
                for lane in range(L):
                    s = exg[lane]
                    e = g * L + lane
                    for j in range(DW // L):
                        sl = pl.ds(j * L, L)
                        rows[e, sl] = rows[e, sl] * s
                return 0
            lax.fori_loop(0, C // L, rowmul, 0)

        pltpu.sync_copy(ipk.at[cbase], idxA)
        pltpu.async_copy(zp_hbm.at[idxA.at[0]], rowsA, gsemA)

        def pair(k, _):
            i0 = 2 * k
            pltpu.sync_copy(ipk.at[cbase + i0 + 1], idxB)
            pltpu.async_copy(zp_hbm.at[idxB.at[0]], rowsB, gsemB)
            pltpu.make_async_copy(zp_hbm.at[idxA.at[0]], rowsA, gsemA).wait()
            compute(idxA, rowsA)
            pltpu.sync_copy(rowsA, u_sh.at[idxA.at[1]], add=True)

            @pl.when(k < n_pairs - 1)
            def _():
                pltpu.sync_copy(ipk.at[cbase + i0 + 2], idxA)
                pltpu.async_copy(zp_hbm.at[idxA.at[0]], rowsA, gsemA)

            pltpu.make_async_copy(zp_hbm.at[idxB.at[0]], rowsB, gsemB).wait()
            compute(idxB, rowsB)
            pltpu.sync_copy(rowsB, u_sh.at[idxB.at[1]], add=True)
            return 0
        lax.fori_loop(0, n_pairs, pair, 0)

        plsc.subcore_barrier()
        ob = sid * rows_per_tile
        pltpu.sync_copy(u_sh.at[pl.ds(ob, rows_per_tile)],
                        out_hbm.at[cid, pl.ds(ob, rows_per_tile)])

    return ek


def _pack_idx(srcp, dstp, tfp, c):
    return jnp.stack([srcp.reshape(-1, c), dstp.reshape(-1, c),
                      tfp.reshape(-1, c)], axis=1)


def _prep(h_src, h_dst, W, Wd, a_s, a_d, etab, a_e, Ns16, Ndt):
    Ns = h_src.shape[0]
    Nd = h_dst.shape[0]
    T = etab.shape[0]

    def body(hs_ref, hd_ref, w_ref, wd_ref, as_ref, ad_ref, te_ref, ae_ref,
             zp_ref, zs_ref, zd_ref, ea_ref, st_ref):
        z = jnp.dot(hs_ref[...], w_ref[...], preferred_element_type=jnp.float32)
        zp_ref[...] = jnp.zeros((Ns16, DW), jnp.float32)
        zp_ref[0:Ns, 0:D] = z
        zp_ref[0:Ns, D:D + 1] = jnp.ones((Ns, 1), jnp.float32)
        zsv = jnp.dot(z, as_ref[...], preferred_element_type=jnp.float32)
        zs_ref[...] = jnp.zeros((Ns16, 1), jnp.float32)
        zs_ref[0:Ns, :] = zsv
        wdv = jnp.dot(wd_ref[...], ad_ref[...], preferred_element_type=jnp.float32)
        zdv = jnp.dot(hd_ref[...], wdv, preferred_element_type=jnp.float32)
        zd_ref[...] = jnp.zeros((Ndt, 1), jnp.float32)
        zd_ref[0:Nd, :] = zdv
        eav = jnp.dot(te_ref[...], ae_ref[...], preferred_element_type=jnp.float32)
        ea_ref[...] = jnp.zeros((1, 16), jnp.float32)
        ea_ref[0:1, 0:T] = jnp.reshape(eav, (1, T))
        m = jnp.max(zsv) + jnp.max(zdv) + jnp.max(eav)
        m = jnp.maximum(m, 0.2 * m)
        st_ref[...] = jnp.full((1, 16), m, jnp.float32)

    zp, zs, zd, ea, st = pl.pallas_call(
        body,
        out_shape=[
            jax.ShapeDtypeStruct((Ns16, DW), jnp.float32),
            jax.ShapeDtypeStruct((Ns16, 1), jnp.float32),
            jax.ShapeDtypeStruct((Ndt, 1), jnp.float32),
            jax.ShapeDtypeStruct((1, 16), jnp.float32),
            jax.ShapeDtypeStruct((1, 16), jnp.float32),
        ],
    )(h_src, h_dst, W, Wd, a_s.reshape(D, 1), a_d.reshape(D, 1), etab,
      a_e.reshape(-1, 1))
    return zp, zs.reshape(-1), zd.reshape(-1), ea.reshape(-1), st.reshape(-1)


def _epilogue(up, h_dst, W1, W2):
    Nd = h_dst.shape[0]
    Nd16 = up.shape[1]
    BR = min(2048, Nd16)
    grid = (pl.cdiv(Nd16, BR),)

    def body(up_ref, hd_ref, w1_ref, w2_ref, out_ref):
        u = up_ref[0] + up_ref[1]
        den = u[:, D:D + 1]
        safe = jnp.where(den > 0, den, 1.0)
        agg = jnp.where(den > 0, u[:, 0:D] / safe, 0.0)
        h = jnp.where(agg > 0, agg, jnp.exp(jnp.minimum(agg, 0.0)) - 1.0)
        hf = jnp.dot(jnp.maximum(jnp.dot(h, w1_ref[...],
                                         preferred_element_type=jnp.float32),
                                 0.0),
                     w2_ref[...], preferred_element_type=jnp.float32)
        out_ref[...] = hd_ref[...] + h + hf

    return pl.pallas_call(
        body,
        grid=grid,
        in_specs=[
            pl.BlockSpec((2, BR, DW), lambda i: (0, i, 0)),
            pl.BlockSpec((BR, D), lambda i: (i, 0)),
            pl.BlockSpec((D, FFN), lambda i: (0, 0)),
            pl.BlockSpec((FFN, D), lambda i: (0, 0)),
        ],
        out_specs=pl.BlockSpec((BR, D), lambda i: (i, 0)),
        out_shape=jax.ShapeDtypeStruct((Nd, D), jnp.float32),
    )(up, h_dst, W1, W2)


def _matmul_tc(x, w):
    def body(x_ref, w_ref, o_ref):
        o_ref[...] = jnp.dot(x_ref[...], w_ref[...],
                             preferred_element_type=jnp.float32)
    return pl.pallas_call(
        body,
        out_shape=jax.ShapeDtypeStruct((x.shape[0], w.shape[1]), jnp.float32),
    )(x, w)


def _head_tc(x, wh, bh):
    def body(x_ref, w_ref, b_ref, o_ref):
        y = jnp.dot(x_ref[...], w_ref[...], preferred_element_type=jnp.float32)
        o_ref[...] = 1.0 / (1.0 + jnp.exp(-(y + b_ref[...])))
    return pl.pallas_call(
        body,
        out_shape=jax.ShapeDtypeStruct((x.shape[0], wh.shape[1]), jnp.float32),
    )(x, wh, bh.reshape(1, -1))


def _gat_layer(h_src, h_dst, ipk, etab, p, edge_k, Ns16, Ndt):
    zp, zs, zd, ea, st = _prep(h_src, h_dst, p['W'], p['Wd'], p['a_s'],
                               p['a_d'], etab, p['a_e'], Ns16, Ndt)
    up = edge_k(ipk, zp, zs, zd, ea, st)
    return _epilogue(up, h_dst, p['W1'], p['W2'])


def kernel(wid, ws_src, ws_dst, tffrac, ww_src, ww_dst, tffrac_ww,
           ss_src, ss_dst, simfrac, sent_init, embed_table, tf_embed,
           sim_embed, W_proj, p_w2s, p_s2w, p_s2s, p_w2w, Wh, bh):
    N_W = wid.shape[0]
    N_S = sent_init.shape[0]
    E_WS = ws_src.shape[0]
    E_WW = ww_src.shape[0]
    E_SS = ss_src.shape[0]
    NW16 = _rup(N_W, 16)
    NS16 = _rup(N_S, 16)
    NW_P = _rup(N_W, 128)
    NS_P = _rup(N_S, 128)

    EP_WS = _rup(E_WS, NW * CHUNK * 2)
    EP_WW = _rup(E_WW, NW * 64 * 2)
    EP_SS = _rup(E_SS, NW * CHUNK * 2)
    i32 = jnp.int32
    ws_s = _pad1(ws_src.astype(i32), EP_WS, 0)
    ws_d = _pad1(ws_dst.astype(i32), EP_WS, NS_P)
    ws_d_rev = _pad1(ws_dst.astype(i32), EP_WS, 0)
    ws_s_rev = _pad1(ws_src.astype(i32), EP_WS, NW_P)
    tf_p = _pad1(tffrac.astype(i32), EP_WS, 0)
    ww_s = _pad1(ww_src.astype(i32), EP_WW, 0)
    ww_d = _pad1(ww_dst.astype(i32), EP_WW, NW_P)
    tfw_p = _pad1(tffrac_ww.astype(i32), EP_WW, 0)
    ss_s = _pad1(ss_src.astype(i32), EP_SS, 0)
    ss_d = _pad1(ss_dst.astype(i32), EP_SS, NS_P)
    sim_p = _pad1(simfrac.astype(i32), EP_SS, 0)

    B = _rup(N_W, NW * 64)
    widp = _pad1(wid.astype(i32), B, 0)
    word_feature = _sc_gather(embed_table, widp, (B // NW) // 64)[:N_W]

    sent_feature = _matmul_tc(sent_init, W_proj)

    ek_w2s = _make_edge_kernel(EP_WS, NW16, NS_P, CHUNK)
    ek_w2w = _make_edge_kernel(EP_WW, NW16, NW_P, 64)
    ek_s2w = _make_edge_kernel(EP_WS, NS16, NW_P, 64)
    ek_s2s = _make_edge_kernel(EP_SS, NS16, NS_P, CHUNK)

    ipk_w2s = _pack_idx(ws_s, ws_d, tf_p, CHUNK)
    ipk_w2w = _pack_idx(ww_s, ww_d, tfw_p, 64)
    ipk_s2w = _pack_idx(ws_d_rev, ws_s_rev, tf_p, 64)
    ipk_s2s = _pack_idx(ss_s, ss_d, sim_p, CHUNK)

    word_state = word_feature
    sent_state = _gat_layer(word_state, sent_feature, ipk_w2s,
                            tf_embed, p_w2s, ek_w2s, NW16, NS_P + 16)
    word_state = _gat_layer(word_state, word_state, ipk_w2w,
                            tf_embed, p_w2w, ek_w2w, NW16, NW_P + 16)
    word_state = _gat_layer(sent_state, word_state, ipk_s2w,
                            tf_embed, p_s2w, ek_s2w, NS16, NW_P + 16)
    sent_state = _gat_layer(word_state, sent_state, ipk_w2s,
                            tf_embed, p_w2s, ek_w2s, NW16, NS_P + 16)
    sent_state = _gat_layer(sent_state, sent_state, ipk_s2s,
                            sim_embed, p_s2s, ek_s2s, NS16, NS_P + 16)

    return _head_tc(sent_state, Wh, bh)


# speedup vs baseline: 1.5753x; 1.0014x over previous
import functools

import jax
import jax.numpy as jnp
from jax import lax
from jax.experimental import pallas as pl
from jax.experimental.pallas import tpu as pltpu
from jax.experimental.pallas import tpu_sc as plsc

NC, NS, L = 2, 16, 16
NW = NC * NS
D = 128
DW = 144
CHUNK = 128
FFN = 512


def _rup(x, m):
    return (x + m - 1) // m * m


def _pad1(x, n, val):
    if x.shape[0] == n:
        return x
    return jnp.concatenate([x, jnp.full((n - x.shape[0],), val, x.dtype)])


@functools.partial(jax.jit, static_argnums=(2,))
def _sc_gather(table, idx, n_chunks):
    B = idx.shape[0]
    bpw = B // NW
    CW = bpw // n_chunks
    Dm = table.shape[1]
    mesh = plsc.VectorSubcoreMesh(core_axis_name="c", subcore_axis_name="s",
                                  num_cores=NC, num_subcores=NS)

    @functools.partial(
        pl.kernel,
        out_type=jax.ShapeDtypeStruct((B, Dm), jnp.float32),
        mesh=mesh,
        compiler_params=pltpu.CompilerParams(use_tc_tiling_on_sc=False, needs_layout_passes=False),
        scratch_types=[
            pltpu.VMEM((CW,), jnp.int32),
            pltpu.VMEM((CW, Dm), jnp.float32),
            pltpu.SemaphoreType.DMA,
        ],
    )
    def gk(tab_hbm, idx_hbm, out_hbm, idx_v, rows_v, sem):
        wid = lax.axis_index("s") * NC + lax.axis_index("c")
        base = wid * bpw
        for i in range(n_chunks):
            off = base + i * CW
            pltpu.sync_copy(idx_hbm.at[pl.ds(off, CW)], idx_v)
            pltpu.async_copy(tab_hbm.at[idx_v], rows_v, sem).wait()
            pltpu.sync_copy(rows_v, out_hbm.at[pl.ds(off, CW)])

    return gk(table, idx)


def _make_edge_kernel(E_pad, Nsrc16, NdP, C):
    E_half = E_pad // NC
    n_t = E_half // NS
    n_chunks = n_t // C
    n_pairs = n_chunks // 2
    R = NdP + 16
    rows_per_tile = NdP // NS
    zr = R // NS
    mesh = plsc.VectorSubcoreMesh(core_axis_name="c", subcore_axis_name="s",
                                  num_cores=NC, num_subcores=NS)

    @functools.partial(
        pl.kernel,
        out_type=jax.ShapeDtypeStruct((NC, NdP, DW), jnp.float32),
        mesh=mesh,
        compiler_params=pltpu.CompilerParams(use_tc_tiling_on_sc=False,
                                             needs_layout_passes=False),
        scratch_types=[
            pltpu.VMEM((Nsrc16,), jnp.float32),
            pltpu.VMEM((NdP + 16,), jnp.float32),
            pltpu.VMEM((16,), jnp.float32),
            pltpu.VMEM((16,), jnp.float32),
            pltpu.VMEM((3, C), jnp.int32),
            pltpu.VMEM((3, C), jnp.int32),
            pltpu.VMEM((C, DW), jnp.float32),
            pltpu.VMEM((C, DW), jnp.float32),
            pltpu.VMEM((C,), jnp.float32),
            pltpu.VMEM((C,), jnp.float32),
            pltpu.VMEM_SHARED((R, DW), jnp.float32),
            pltpu.SemaphoreType.DMA,
            pltpu.SemaphoreType.DMA,
        ],
    )
    def ek(ipk, zp_hbm, zs_hbm, zd_hbm, ea_hbm, stab_hbm, out_hbm,
           zs_tab, zd_tab, ea_tab, stab_v, idxA, idxB, rowsA, rowsB,
           exA, exB, u_sh, gsemA, gsemB):
        cid = lax.axis_index("c")
        sid = lax.axis_index("s")

        pltpu.sync_copy(zs_hbm, zs_tab)
        pltpu.sync_copy(zd_hbm, zd_tab)
        pltpu.sync_copy(ea_hbm, ea_tab)
        pltpu.sync_copy(stab_hbm, stab_v)
        stab = stab_v[...]

        def zrow(i, _):
            for j in range(DW // L):
                rowsA[i, pl.ds(j * L, L)] = jnp.zeros((L,), jnp.float32)
            return 0
        lax.fori_loop(0, C, zrow, 0)
        zbase = sid * zr
        off = 0
        while off < zr:
            n = min(C, zr - off)
            pltpu.sync_copy(rowsA.at[pl.ds(0, n)],
                            u_sh.at[pl.ds(zbase + off, n)])
            off += n
        plsc.subcore_barrier()

        cbase = (cid * NS + sid) * n_chunks
        ohv = jnp.where(lax.iota(jnp.int32, 16) == 0, 1.0, 0.0)

        def grp_phase(idx, ex_v):
            def grp(g, _):
                sl = pl.ds(g * L, L)
                lg = (plsc.load_gather(zs_tab, [idx[0, sl]])
                      + plsc.load_gather(zd_tab, [idx[1, sl]])
                      + plsc.load_gather(ea_tab, [idx[2, sl]]))
                lg = jnp.maximum(lg, 0.2 * lg)
                ex_v[sl] = jnp.exp(lg - stab)
                return 0
            lax.fori_loop(0, C // L, grp, 0)

        def rowmul_phase(ex_v, rows):
            def rowmul(g, _):
                exg = ex_v[pl.ds(g * L, L)]
                for lane in range(L):
                    s = exg[lane]
                    e = g * L + lane
                    for j in range(DW // L - 1):
                        sl = pl.ds(j * L, L)
                        rows[e, sl] = rows[e, sl] * s
                    rows[e, pl.ds(D, L)] = s * ohv
                return 0
            lax.fori_loop(0, C // L, rowmul, 0)

        pltpu.sync_copy(ipk.at[cbase], idxA)
        pltpu.async_copy(zp_hbm.at[idxA.at[0]], rowsA, gsemA)
        grp_phase(idxA, exA)

        def pair(k, _):
            i0 = 2 * k
            pltpu.sync_copy(ipk.at[cbase + i0 + 1], idxB)
            pltpu.async_copy(zp_hbm.at[idxB.at[0]], rowsB, gsemB)
            grp_phase(idxB, exB)
            pltpu.make_async_copy(zp_hbm.at[idxA.at[0]], rowsA, gsemA).wait()
            rowmul_phase(exA, rowsA)
            pltpu.sync_copy(rowsA, u_sh.at[idxA.at[1]], add=True)

            @pl.when(k < n_pairs - 1)
            def _():
                pltpu.sync_copy(ipk.at[cbase + i0 + 2], idxA)
                pltpu.async_copy(zp_hbm.at[idxA.at[0]], rowsA, gsemA)
                grp_phase(idxA, exA)

            pltpu.make_async_copy(zp_hbm.at[idxB.at[0]], rowsB, gsemB).wait()
            rowmul_phase(exB, rowsB)
            pltpu.sync_copy(rowsB, u_sh.at[idxB.at[1]], add=True)
            return 0
        lax.fori_loop(0, n_pairs, pair, 0)

        plsc.subcore_barrier()
        ob = sid * rows_per_tile
        pltpu.sync_copy(u_sh.at[pl.ds(ob, rows_per_tile)],
                        out_hbm.at[cid, pl.ds(ob, rows_per_tile)])

    return ek


def _pack_idx(srcp, dstp, tfp, c):
    return jnp.stack([srcp.reshape(-1, c), dstp.reshape(-1, c),
                      tfp.reshape(-1, c)], axis=1)


def _prep(h_src, h_dst, W, Wd, a_s, a_d, etab, a_e, Ns16, Ndt):
    Ns = h_src.shape[0]
    Nd = h_dst.shape[0]
    T = etab.shape[0]

    def body(hs_ref, hd_ref, w_ref, wd_ref, as_ref, ad_ref, te_ref, ae_ref,
             zp_ref, zs_ref, zd_ref, ea_ref, st_ref):
        z = jnp.dot(hs_ref[...], w_ref[...], preferred_element_type=jnp.float32)
        zp_ref[...] = jnp.zeros((Ns16, DW), jnp.float32)
        zp_ref[0:Ns, 0:D] = z
        zp_ref[0:Ns, D:D + 1] = jnp.ones((Ns, 1), jnp.float32)
        zsv = jnp.dot(z, as_ref[...], preferred_element_type=jnp.float32)
        zs_ref[...] = jnp.zeros((Ns16, 1), jnp.float32)
        zs_ref[0:Ns, :] = zsv
        wdv = jnp.dot(wd_ref[...], ad_ref[...], preferred_element_type=jnp.float32)
        zdv = jnp.dot(hd_ref[...], wdv, preferred_element_type=jnp.float32)
        zd_ref[...] = jnp.zeros((Ndt, 1), jnp.float32)
        zd_ref[0:Nd, :] = zdv
        eav = jnp.dot(te_ref[...], ae_ref[...], preferred_element_type=jnp.float32)
        ea_ref[...] = jnp.zeros((1, 16), jnp.float32)
        ea_ref[0:1, 0:T] = jnp.reshape(eav, (1, T))
        m = jnp.max(zsv) + jnp.max(zdv) + jnp.max(eav)
        m = jnp.maximum(m, 0.2 * m)
        st_ref[...] = jnp.full((1, 16), m, jnp.float32)

    zp, zs, zd, ea, st = pl.pallas_call(
        body,
        out_shape=[
            jax.ShapeDtypeStruct((Ns16, DW), jnp.float32),
            jax.ShapeDtypeStruct((Ns16, 1), jnp.float32),
            jax.ShapeDtypeStruct((Ndt, 1), jnp.float32),
            jax.ShapeDtypeStruct((1, 16), jnp.float32),
            jax.ShapeDtypeStruct((1, 16), jnp.float32),
        ],
    )(h_src, h_dst, W, Wd, a_s.reshape(D, 1), a_d.reshape(D, 1), etab,
      a_e.reshape(-1, 1))
    return zp, zs.reshape(-1), zd.reshape(-1), ea.reshape(-1), st.reshape(-1)


def _epilogue(up, h_dst, W1, W2):
    Nd = h_dst.shape[0]
    Nd16 = up.shape[1]
    BR = min(2048, Nd16)
    grid = (pl.cdiv(Nd16, BR),)

    def body(up_ref, hd_ref, w1_ref, w2_ref, out_ref):
        u = up_ref[0] + up_ref[1]
        den = u[:, D:D + 1]
        safe = jnp.where(den > 0, den, 1.0)
        agg = jnp.where(den > 0, u[:, 0:D] / safe, 0.0)
        h = jnp.where(agg > 0, agg, jnp.exp(jnp.minimum(agg, 0.0)) - 1.0)
        hf = jnp.dot(jnp.maximum(jnp.dot(h, w1_ref[...],
                                         preferred_element_type=jnp.float32),
                                 0.0),
                     w2_ref[...], preferred_element_type=jnp.float32)
        out_ref[...] = hd_ref[...] + h + hf

    return pl.pallas_call(
        body,
        grid=grid,
        in_specs=[
            pl.BlockSpec((2, BR, DW), lambda i: (0, i, 0)),
            pl.BlockSpec((BR, D), lambda i: (i, 0)),
            pl.BlockSpec((D, FFN), lambda i: (0, 0)),
            pl.BlockSpec((FFN, D), lambda i: (0, 0)),
        ],
        out_specs=pl.BlockSpec((BR, D), lambda i: (i, 0)),
        out_shape=jax.ShapeDtypeStruct((Nd, D), jnp.float32),
    )(up, h_dst, W1, W2)


def _matmul_tc(x, w):
    def body(x_ref, w_ref, o_ref):
        o_ref[...] = jnp.dot(x_ref[...], w_ref[...],
                             preferred_element_type=jnp.float32)
    return pl.pallas_call(
        body,
        out_shape=jax.ShapeDtypeStruct((x.shape[0], w.shape[1]), jnp.float32),
    )(x, w)


def _head_tc(x, wh, bh):
    def body(x_ref, w_ref, b_ref, o_ref):
        y = jnp.dot(x_ref[...], w_ref[...], preferred_element_type=jnp.float32)
        o_ref[...] = 1.0 / (1.0 + jnp.exp(-(y + b_ref[...])))
    return pl.pallas_call(
        body,
        out_shape=jax.ShapeDtypeStruct((x.shape[0], wh.shape[1]), jnp.float32),
    )(x, wh, bh.reshape(1, -1))


def _gat_layer(h_src, h_dst, ipk, etab, p, edge_k, Ns16, Ndt):
    zp, zs, zd, ea, st = _prep(h_src, h_dst, p['W'], p['Wd'], p['a_s'],
                               p['a_d'], etab, p['a_e'], Ns16, Ndt)
    up = edge_k(ipk, zp, zs, zd, ea, st)
    return _epilogue(up, h_dst, p['W1'], p['W2'])


def kernel(wid, ws_src, ws_dst, tffrac, ww_src, ww_dst, tffrac_ww,
           ss_src, ss_dst, simfrac, sent_init, embed_table, tf_embed,
           sim_embed, W_proj, p_w2s, p_s2w, p_s2s, p_w2w, Wh, bh):
    N_W = wid.shape[0]
    N_S = sent_init.shape[0]
    E_WS = ws_src.shape[0]
    E_WW = ww_src.shape[0]
    E_SS = ss_src.shape[0]
    NW16 = _rup(N_W, 16)
    NS16 = _rup(N_S, 16)
    NW_P = _rup(N_W, 128)
    NS_P = _rup(N_S, 128)

    EP_WS = _rup(E_WS, NW * CHUNK * 2)
    EP_WW = _rup(E_WW, NW * 64 * 2)
    EP_SS = _rup(E_SS, NW * CHUNK * 2)
    i32 = jnp.int32
    ws_s = _pad1(ws_src.astype(i32), EP_WS, 0)
    ws_d = _pad1(ws_dst.astype(i32), EP_WS, NS_P)
    ws_d_rev = _pad1(ws_dst.astype(i32), EP_WS, 0)
    ws_s_rev = _pad1(ws_src.astype(i32), EP_WS, NW_P)
    tf_p = _pad1(tffrac.astype(i32), EP_WS, 0)
    ww_s = _pad1(ww_src.astype(i32), EP_WW, 0)
    ww_d = _pad1(ww_dst.astype(i32), EP_WW, NW_P)
    tfw_p = _pad1(tffrac_ww.astype(i32), EP_WW, 0)
    ss_s = _pad1(ss_src.astype(i32), EP_SS, 0)
    ss_d = _pad1(ss_dst.astype(i32), EP_SS, NS_P)
    sim_p = _pad1(simfrac.astype(i32), EP_SS, 0)

    B = _rup(N_W, NW * 64)
    widp = _pad1(wid.astype(i32), B, 0)
    word_feature = _sc_gather(embed_table, widp, (B // NW) // 64)[:N_W]

    sent_feature = _matmul_tc(sent_init, W_proj)

    ek_w2s = _make_edge_kernel(EP_WS, NW16, NS_P, CHUNK)
    ek_w2w = _make_edge_kernel(EP_WW, NW16, NW_P, 64)
    ek_s2w = _make_edge_kernel(EP_WS, NS16, NW_P, 64)
    ek_s2s = _make_edge_kernel(EP_SS, NS16, NS_P, CHUNK)

    ipk_w2s = _pack_idx(ws_s, ws_d, tf_p, CHUNK)
    ipk_w2w = _pack_idx(ww_s, ww_d, tfw_p, 64)
    ipk_s2w = _pack_idx(ws_d_rev, ws_s_rev, tf_p, 64)
    ipk_s2s = _pack_idx(ss_s, ss_d, sim_p, CHUNK)

    word_state = word_feature
    sent_state = _gat_layer(word_state, sent_feature, ipk_w2s,
                            tf_embed, p_w2s, ek_w2s, NW16, NS_P + 16)
    word_state = _gat_layer(word_state, word_state, ipk_w2w,
                            tf_embed, p_w2w, ek_w2w, NW16, NW_P + 16)
    word_state = _gat_layer(sent_state, word_state, ipk_s2w,
                            tf_embed, p_s2w, ek_s2w, NS16, NW_P + 16)
    sent_state = _gat_layer(word_state, sent_state, ipk_w2s,
                            tf_embed, p_w2s, ek_w2s, NW16, NS_P + 16)
    sent_state = _gat_layer(sent_state, sent_state, ipk_s2s,
                            sim_embed, p_s2s, ek_s2s, NS16, NS_P + 16)

    return _head_tc(sent_state, Wh, bh)
